# baseline (speedup 1.0000x reference)
import functools

import jax
import jax.numpy as jnp
from jax import lax
from jax.experimental import pallas as pl
from jax.experimental.pallas import tpu as pltpu
from jax.experimental.pallas import tpu_sc as plsc

N_FACTORS = 32
NEG_N = 2

NC = 2
NS = 16
LANES = 16


def _sc_scores(w1t, w2t, idx_t, idx_w2, batch):
    n = w1t.shape[1]
    b_per_w = batch // NS
    ch_len = 1024
    nch_t = b_per_w // ch_len
    nch_w2 = 3 * b_per_w // ch_len
    d_per_c = N_FACTORS // NC
    piece = (n // (NS * 8)) * 8
    last = n - (NS - 1) * piece
    mesh = plsc.VectorSubcoreMesh(core_axis_name="c", subcore_axis_name="s")

    @functools.partial(
        pl.kernel,
        out_type=[
            jax.ShapeDtypeStruct((NC, batch), jnp.float32),
            jax.ShapeDtypeStruct((NC, batch), jnp.float32),
        ],
        mesh=mesh,
        scratch_types=[
            pltpu.VMEM_SHARED((1, n), jnp.float32),
            pltpu.VMEM((ch_len,), jnp.int32),
            pltpu.VMEM((ch_len,), jnp.int32),
            pltpu.VMEM((ch_len,), jnp.float32),
            pltpu.VMEM((ch_len,), jnp.float32),
            pltpu.VMEM((b_per_w,), jnp.bfloat16),
            pltpu.VMEM((b_per_w,), jnp.float32),
            pltpu.VMEM((b_per_w,), jnp.float32),
            pltpu.SemaphoreType.DMA,
            pltpu.SemaphoreType.DMA,
            pltpu.SemaphoreType.DMA,
            pltpu.SemaphoreType.DMA,
            pltpu.SemaphoreType.DMA,
        ],
        compiler_params=pltpu.CompilerParams(
            needs_layout_passes=False, use_tc_tiling_on_sc=True),
    )
    def k(w1t_hbm, w2t_hbm, it_hbm, iw2_hbm, spp_hbm, spn_hbm,
          row_sh, g0, g1, v0, v1, u_t, accp_v, accn_v,
          sem_stage, sre0, sre1, sga0, sga1):
        c = lax.axis_index("c")
        s = lax.axis_index("s")

        def zero_body(i, carry):
            accp_v[pl.ds(i * LANES, LANES)] = jnp.zeros((LANES,), jnp.float32)
            accn_v[pl.ds(i * LANES, LANES)] = jnp.zeros((LANES,), jnp.float32)
            return carry
        lax.fori_loop(0, b_per_w // LANES, zero_body, 0, unroll=False)

        def stage_row(tbl_hbm, d):
            @pl.when(s < NS - 1)
            def _():
                pltpu.async_copy(
                    tbl_hbm.at[pl.ds(d, 1), pl.ds(s * piece, piece)],
                    row_sh.at[:, pl.ds(s * piece, piece)], sem_stage).wait()

            @pl.when(s == NS - 1)
            def _():
                pltpu.async_copy(
                    tbl_hbm.at[pl.ds(d, 1), pl.ds((NS - 1) * piece, last)],
                    row_sh.at[:, pl.ds((NS - 1) * piece, last)],
                    sem_stage).wait()
            plsc.subcore_barrier()

        def gather_phase(idx_hbm, nch, consume):
            def reload(cc, g, sem):
                return pltpu.make_async_copy(
                    idx_hbm.at[s, pl.ds(cc * ch_len, ch_len)], g, sem)

            def gather(g, v, sem):
                return pltpu.make_async_copy(row_sh.at[0].at[g], v, sem)

            def step(cc, gc, vc, go, vo, srec, sreo, sgac, sgao):
                reload(cc, gc, srec).wait()
                gather(gc, vc, sgac).start()

                @pl.when(cc > 0)
                def _():
                    gather(go, vo, sgao).wait()
                    consume(cc - 1, vo)

                @pl.when(cc + 1 < nch)
                def _():
                    reload(cc + 1, go, sreo).start()

            reload(0, g0, sre0).start()

            def chunk_body(cc, carry):
                @pl.when(cc % 2 == 0)
                def _():
                    step(cc, g0, v0, g1, v1, sre0, sre1, sga0, sga1)

                @pl.when(cc % 2 == 1)
                def _():
                    step(cc, g1, v1, g0, v0, sre1, sre0, sga1, sga0)
                return carry
            lax.fori_loop(0, nch, chunk_body, 0, unroll=False)
            gather(g1, v1, sga1).wait()
            consume(nch - 1, v1)

        def consume_t(cc, v):
            def body(j, carry):
                a = v[pl.ds(2 * j * LANES, LANES)]
                b = v[pl.ds((2 * j + 1) * LANES, LANES)]
                u_t[pl.ds(cc * ch_len + 2 * j * LANES, 2 * LANES)] = (
                    plsc.pack(a, b, format=plsc.PackFormat.INTERLEAVED))
                return carry
            lax.fori_loop(0, ch_len // (2 * LANES), body, 0, unroll=False)

        def consume_w2(cc, v):
            boff = (cc % nch_t) * ch_len

            @pl.when(cc < nch_t)
            def _():
                def body(j, carry):
                    ua, ub = plsc.unpack(
                        u_t[pl.ds(boff + 2 * j * LANES, 2 * LANES)],
                        format=plsc.PackFormat.INTERLEAVED)
                    sla = pl.ds(boff + 2 * j * LANES, LANES)
                    slb = pl.ds(boff + (2 * j + 1) * LANES, LANES)
                    accp_v[sla] = accp_v[sla] + ua * v[pl.ds(2 * j * LANES, LANES)]
                    accp_v[slb] = accp_v[slb] + ub * v[pl.ds((2 * j + 1) * LANES, LANES)]
                    return carry
                lax.fori_loop(0, ch_len // (2 * LANES), body, 0, unroll=False)

            @pl.when(cc >= nch_t)
            def _():
                def body(j, carry):
                    ua, ub = plsc.unpack(
                        u_t[pl.ds(boff + 2 * j * LANES, 2 * LANES)],
                        format=plsc.PackFormat.INTERLEAVED)
                    sla = pl.ds(boff + 2 * j * LANES, LANES)
                    slb = pl.ds(boff + (2 * j + 1) * LANES, LANES)
                    accn_v[sla] = accn_v[sla] - ua * v[pl.ds(2 * j * LANES, LANES)]
                    accn_v[slb] = accn_v[slb] - ub * v[pl.ds((2 * j + 1) * LANES, LANES)]
                    return carry
                lax.fori_loop(0, ch_len // (2 * LANES), body, 0, unroll=False)

        def d_body(dd, carry):
            d = c * d_per_c + dd
            stage_row(w1t_hbm, d)
            gather_phase(it_hbm, nch_t, consume_t)
            plsc.subcore_barrier()
            stage_row(w2t_hbm, d)
            gather_phase(iw2_hbm, nch_w2, consume_w2)
            plsc.subcore_barrier()
            return carry

        lax.fori_loop(0, d_per_c, d_body, 0, unroll=False)

        base = s * b_per_w
        pltpu.sync_copy(accp_v, spp_hbm.at[c, pl.ds(base, b_per_w)])
        pltpu.sync_copy(accn_v, spn_hbm.at[c, pl.ds(base, b_per_w)])

    return k(w1t, w2t, idx_t, idx_w2)


def _tc_loss_body(sp_ref, sn_ref, out_ref):
    sp = sp_ref[0] + sp_ref[1]
    sn = sn_ref[0] + sn_ref[1]
    lp = jnp.minimum(sp, 0.0) - jnp.log1p(jnp.exp(-jnp.abs(sp)))
    ln = jnp.minimum(sn, 0.0) - jnp.log1p(jnp.exp(-jnp.abs(sn)))
    total = jnp.sum(-lp - ln)
    out_ref[0, 0] = total / sp.size


def kernel(target, pos, W1, W2):
    batch = target.shape[0]
    n_aids = W2.shape[0]

    neg = jax.random.randint(jax.random.key(42), (batch, NEG_N), 0, n_aids)

    bw = batch // NS
    idx_t = target.reshape(NS, bw).astype(jnp.int32)
    idx_w2 = jnp.concatenate(
        [pos.reshape(NS, bw).astype(jnp.int32),
         neg[:, 0].reshape(NS, bw).astype(jnp.int32),
         neg[:, 1].reshape(NS, bw).astype(jnp.int32)], axis=1)

    spp, spn = _sc_scores(W1.T, W2.T, idx_t, idx_w2, batch)

    rows = batch // 128
    loss = pl.pallas_call(
        _tc_loss_body,
        out_shape=jax.ShapeDtypeStruct((1, 1), jnp.float32),
        out_specs=pl.BlockSpec(memory_space=pltpu.SMEM),
    )(spp.reshape(NC, rows, 128), spn.reshape(NC, rows, 128))
    return loss[0, 0]


# speedup vs baseline: 1.1912x; 1.1912x over previous
import functools

import jax
import jax.numpy as jnp
from jax import lax
from jax.experimental import pallas as pl
from jax.experimental.pallas import tpu as pltpu
from jax.experimental.pallas import tpu_sc as plsc

N_FACTORS = 32
NEG_N = 2

NC = 2
NS = 16
LANES = 16


def _sc_scores(w1t, w2t, idx_t, idx_w2, batch):
    n = w1t.shape[1]
    b_per_w = batch // NS
    ch_len = 1024
    nch_t = b_per_w // ch_len
    nch_w2 = 3 * b_per_w // ch_len
    d_per_c = N_FACTORS // NC
    piece = (n // (NS * 8)) * 8
    last = n - (NS - 1) * piece
    mesh = plsc.VectorSubcoreMesh(core_axis_name="c", subcore_axis_name="s")

    @functools.partial(
        pl.kernel,
        out_type=[
            jax.ShapeDtypeStruct((NC, batch), jnp.float32),
            jax.ShapeDtypeStruct((NC, batch), jnp.float32),
        ],
        mesh=mesh,
        scratch_types=[
            pltpu.VMEM_SHARED((1, n), jnp.float32),
            pltpu.VMEM((ch_len,), jnp.int32),
            pltpu.VMEM((ch_len,), jnp.int32),
            pltpu.VMEM((ch_len,), jnp.float32),
            pltpu.VMEM((ch_len,), jnp.float32),
            pltpu.VMEM((b_per_w,), jnp.bfloat16),
            pltpu.VMEM((b_per_w,), jnp.float32),
            pltpu.VMEM((b_per_w,), jnp.float32),
            pltpu.SemaphoreType.DMA,
            pltpu.SemaphoreType.DMA,
            pltpu.SemaphoreType.DMA,
            pltpu.SemaphoreType.DMA,
            pltpu.SemaphoreType.DMA,
        ],
        compiler_params=pltpu.CompilerParams(
            needs_layout_passes=False, use_tc_tiling_on_sc=True),
    )
    def k(w1t_hbm, w2t_hbm, it_hbm, iw2_hbm, spp_hbm, spn_hbm,
          row_sh, g0, g1, v0, v1, u_t, accp_v, accn_v,
          sem_stage, sre0, sre1, sga0, sga1):
        c = lax.axis_index("c")
        s = lax.axis_index("s")

        def zero_body(i, carry):
            accp_v[pl.ds(i * LANES, LANES)] = jnp.zeros((LANES,), jnp.float32)
            accn_v[pl.ds(i * LANES, LANES)] = jnp.zeros((LANES,), jnp.float32)
            return carry
        lax.fori_loop(0, b_per_w // LANES, zero_body, 0, unroll=False)

        def stage_row(tbl_hbm, d):
            @pl.when(s < NS - 1)
            def _():
                pltpu.async_copy(
                    tbl_hbm.at[pl.ds(d, 1), pl.ds(s * piece, piece)],
                    row_sh.at[:, pl.ds(s * piece, piece)], sem_stage).wait()

            @pl.when(s == NS - 1)
            def _():
                pltpu.async_copy(
                    tbl_hbm.at[pl.ds(d, 1), pl.ds((NS - 1) * piece, last)],
                    row_sh.at[:, pl.ds((NS - 1) * piece, last)],
                    sem_stage).wait()
            plsc.subcore_barrier()

        def gather_phase(idx_hbm, nch, consume, prestarted, next_idx_hbm):
            def reload(cc, g, sem):
                return pltpu.make_async_copy(
                    idx_hbm.at[s, pl.ds(cc * ch_len, ch_len)], g, sem)

            def gather(g, v, sem):
                return pltpu.make_async_copy(row_sh.at[0].at[g], v, sem)

            def step(cc, gc, vc, go, vo, srec, sreo, sgac, sgao):
                reload(cc, gc, srec).wait()
                gather(gc, vc, sgac).start()

                @pl.when(cc > 0)
                def _():
                    gather(go, vo, sgao).wait()

                    @pl.when(cc + 1 < nch)
                    def _():
                        reload(cc + 1, go, sreo).start()
                    consume(cc - 1, vo)

                @pl.when(jnp.logical_and(cc == 0, cc + 1 < nch))
                def _():
                    reload(cc + 1, go, sreo).start()

            del prestarted

            def chunk_body(cc, carry):
                @pl.when(cc % 2 == 0)
                def _():
                    step(cc, g0, v0, g1, v1, sre0, sre1, sga0, sga1)

                @pl.when(cc % 2 == 1)
                def _():
                    step(cc, g1, v1, g0, v0, sre1, sre0, sga1, sga0)
                return carry
            lax.fori_loop(0, nch, chunk_body, 0, unroll=False)
            gather(g1, v1, sga1).wait()
            pltpu.make_async_copy(
                next_idx_hbm.at[s, pl.ds(0, ch_len)], g0, sre0).start()
            consume(nch - 1, v1)

        def consume_t(cc, v):
            def body(j, carry):
                a = v[pl.ds(2 * j * LANES, LANES)]
                b = v[pl.ds((2 * j + 1) * LANES, LANES)]
                u_t[pl.ds(cc * ch_len + 2 * j * LANES, 2 * LANES)] = (
                    plsc.pack(a, b, format=plsc.PackFormat.INTERLEAVED))
                return carry
            lax.fori_loop(0, ch_len // (2 * LANES), body, 0, unroll=False)

        def consume_w2(cc, v):
            boff = (cc % nch_t) * ch_len

            @pl.when(cc < nch_t)
            def _():
                def body(j, carry):
                    ua, ub = plsc.unpack(
                        u_t[pl.ds(boff + 2 * j * LANES, 2 * LANES)],
                        format=plsc.PackFormat.INTERLEAVED)
                    sla = pl.ds(boff + 2 * j * LANES, LANES)
                    slb = pl.ds(boff + (2 * j + 1) * LANES, LANES)
                    accp_v[sla] = accp_v[sla] + ua * v[pl.ds(2 * j * LANES, LANES)]
                    accp_v[slb] = accp_v[slb] + ub * v[pl.ds((2 * j + 1) * LANES, LANES)]
                    return carry
                lax.fori_loop(0, ch_len // (2 * LANES), body, 0, unroll=False)

            @pl.when(cc >= nch_t)
            def _():
                def body(j, carry):
                    ua, ub = plsc.unpack(
                        u_t[pl.ds(boff + 2 * j * LANES, 2 * LANES)],
                        format=plsc.PackFormat.INTERLEAVED)
                    sla = pl.ds(boff + 2 * j * LANES, LANES)
                    slb = pl.ds(boff + (2 * j + 1) * LANES, LANES)
                    accn_v[sla] = accn_v[sla] - ua * v[pl.ds(2 * j * LANES, LANES)]
                    accn_v[slb] = accn_v[slb] - ub * v[pl.ds((2 * j + 1) * LANES, LANES)]
                    return carry
                lax.fori_loop(0, ch_len // (2 * LANES), body, 0, unroll=False)

        pltpu.make_async_copy(
            it_hbm.at[s, pl.ds(0, ch_len)], g0, sre0).start()

        def d_body(dd, carry):
            d = c * d_per_c + dd
            stage_row(w1t_hbm, d)
            gather_phase(it_hbm, nch_t, consume_t, True, iw2_hbm)
            plsc.subcore_barrier()
            stage_row(w2t_hbm, d)
            gather_phase(iw2_hbm, nch_w2, consume_w2, True, it_hbm)
            plsc.subcore_barrier()
            return carry

        lax.fori_loop(0, d_per_c, d_body, 0, unroll=False)

        base = s * b_per_w
        pltpu.sync_copy(accp_v, spp_hbm.at[c, pl.ds(base, b_per_w)])
        pltpu.sync_copy(accn_v, spn_hbm.at[c, pl.ds(base, b_per_w)])

    return k(w1t, w2t, idx_t, idx_w2)


def _tc_loss_body(sp_ref, sn_ref, out_ref):
    sp = sp_ref[0] + sp_ref[1]
    sn = sn_ref[0] + sn_ref[1]
    lp = jnp.minimum(sp, 0.0) - jnp.log1p(jnp.exp(-jnp.abs(sp)))
    ln = jnp.minimum(sn, 0.0) - jnp.log1p(jnp.exp(-jnp.abs(sn)))
    total = jnp.sum(-lp - ln)
    out_ref[0, 0] = total / sp.size


def kernel(target, pos, W1, W2):
    batch = target.shape[0]
    n_aids = W2.shape[0]

    neg = jax.random.randint(jax.random.key(42), (batch, NEG_N), 0, n_aids)

    bw = batch // NS
    idx_t = target.reshape(NS, bw).astype(jnp.int32)
    idx_w2 = jnp.concatenate(
        [pos.reshape(NS, bw).astype(jnp.int32),
         neg[:, 0].reshape(NS, bw).astype(jnp.int32),
         neg[:, 1].reshape(NS, bw).astype(jnp.int32)], axis=1)

    spp, spn = _sc_scores(W1.T, W2.T, idx_t, idx_w2, batch)

    rows = batch // 128
    loss = pl.pallas_call(
        _tc_loss_body,
        out_shape=jax.ShapeDtypeStruct((1, 1), jnp.float32),
        out_specs=pl.BlockSpec(memory_space=pltpu.SMEM),
    )(spp.reshape(NC, rows, 128), spn.reshape(NC, rows, 128))
    return loss[0, 0]
